# Initial kernel scaffold; baseline (speedup 1.0000x reference)
import jax
import jax.numpy as jnp
from jax.experimental import pallas as pl


def kernel(hidden_states, Wg, W1, W2, W3):
    raise NotImplementedError("write your pallas kernel here")


# speedup vs baseline: 1.1086x; 1.1086x over previous
import jax
import jax.numpy as jnp
from jax.experimental import pallas as pl
from jax.experimental.pallas import tpu as pltpu

HIDDEN = 768
FFN = 3072
E = 8
TOPK = 2
N_TOK = 1024
TB = 256


def _router_body(x_ref, wg_ref, we_ref):
    x = x_ref[...]
    logits = jax.lax.dot_general(
        x, wg_ref[...], (((1,), (1,)), ((), ())),
        preferred_element_type=jnp.float32)
    m = jax.nn.softmax(logits, axis=-1)
    i1 = jnp.argmax(m, axis=-1)
    col = jax.lax.broadcasted_iota(jnp.int32, m.shape, 1)
    is1 = col == i1[:, None]
    w1 = jnp.max(m, axis=-1)
    m2 = jnp.where(is1, -jnp.inf, m)
    i2 = jnp.argmax(m2, axis=-1)
    is2 = col == i2[:, None]
    w2 = jnp.max(m2, axis=-1)
    denom = (w1 + w2)[:, None]
    we = jnp.where(is1, w1[:, None], jnp.where(is2, w2[:, None], 0.0)) / denom
    we_ref[...] = we.astype(jnp.float32)


def _moe_body(x_ref, we_ref, w1_ref, w2_ref, w3_ref, out_ref):
    e = pl.program_id(1)
    x = x_ref[...]
    h1 = jax.lax.dot_general(
        x, w1_ref[0], (((1,), (1,)), ((), ())),
        preferred_element_type=jnp.float32)
    h1 = jnp.maximum(h1, 0.0)
    h2 = jax.lax.dot_general(
        h1, w2_ref[0], (((1,), (1,)), ((), ())),
        preferred_element_type=jnp.float32)
    h2 = jnp.maximum(h2, 0.0)
    y = jax.lax.dot_general(
        h2, w3_ref[0], (((1,), (1,)), ((), ())),
        preferred_element_type=jnp.float32)
    we = we_ref[...]
    col = jax.lax.broadcasted_iota(jnp.int32, we.shape, 1)
    wcol = jnp.sum(jnp.where(col == e, we, 0.0), axis=1, keepdims=True)
    y = y * wcol

    @pl.when(e == 0)
    def _init():
        out_ref[...] = y

    @pl.when(e != 0)
    def _acc():
        out_ref[...] += y


def kernel(hidden_states, Wg, W1, W2, W3):
    b, c, h, w = hidden_states.shape
    x = jnp.transpose(hidden_states, (0, 2, 3, 1)).reshape(-1, c)

    we = pl.pallas_call(
        _router_body,
        out_shape=jax.ShapeDtypeStruct((N_TOK, E), jnp.float32),
    )(x, Wg)

    out = pl.pallas_call(
        _moe_body,
        grid=(N_TOK // TB, E),
        in_specs=[
            pl.BlockSpec((TB, HIDDEN), lambda t, e: (t, 0)),
            pl.BlockSpec((TB, E), lambda t, e: (t, 0)),
            pl.BlockSpec((1, HIDDEN, HIDDEN), lambda t, e: (e, 0, 0)),
            pl.BlockSpec((1, HIDDEN, HIDDEN), lambda t, e: (e, 0, 0)),
            pl.BlockSpec((1, FFN, HIDDEN), lambda t, e: (e, 0, 0)),
        ],
        out_specs=pl.BlockSpec((TB, FFN), lambda t, e: (t, 0)),
        out_shape=jax.ShapeDtypeStruct((N_TOK, FFN), jnp.float32),
        compiler_params=pltpu.CompilerParams(
            dimension_semantics=("arbitrary", "arbitrary"),
        ),
    )(x, we, W1, W2, W3)

    out = out.reshape(b, h, w, FFN)
    return jnp.transpose(out, (0, 3, 1, 2))
